# baseline (speedup 1.0000x reference)
import functools

import jax
import jax.numpy as jnp
from jax import lax
from jax.experimental import pallas as pl
from jax.experimental.pallas import tpu as pltpu
from jax.experimental.pallas import tpu_sc as plsc

DIM = 64
LANES = 16
CHUNK = 128


def kernel(u, i, j, user_factors, item_factors, item_biases):
    B = u.shape[0]
    info = plsc.get_sparse_core_info()
    nw = info.num_cores * info.num_subcores
    bpw = B // nw
    n_chunks = bpw // CHUNK

    mesh = plsc.VectorSubcoreMesh(core_axis_name="c", subcore_axis_name="s")

    @functools.partial(
        pl.kernel,
        mesh=mesh,
        out_type=jax.ShapeDtypeStruct((B,), jnp.float32),
        compiler_params=pltpu.CompilerParams(use_tc_tiling_on_sc=False),
        scratch_types=[
            pltpu.VMEM((bpw,), jnp.int32),
            pltpu.VMEM((bpw,), jnp.int32),
            pltpu.VMEM((bpw,), jnp.int32),
            pltpu.VMEM((bpw, DIM), jnp.float32),
            pltpu.VMEM((bpw, DIM), jnp.float32),
            pltpu.VMEM((bpw, DIM), jnp.float32),
            pltpu.VMEM((bpw,), jnp.float32),
            pltpu.VMEM((bpw,), jnp.float32),
            pltpu.VMEM((bpw,), jnp.float32),
            pltpu.SemaphoreType.DMA,
        ],
    )
    def sc_kernel(u_hbm, i_hbm, j_hbm, uf_hbm, if_hbm, ib_hbm, out_hbm,
                  u_idx, i_idx, j_idx, u_rows, i_rows, j_rows, bi_v, bj_v,
                  out_v, sem):
        wid = lax.axis_index("s") * info.num_cores + lax.axis_index("c")
        base = wid * bpw

        pltpu.sync_copy(u_hbm.at[pl.ds(base, bpw)], u_idx)
        pltpu.sync_copy(i_hbm.at[pl.ds(base, bpw)], i_idx)
        pltpu.sync_copy(j_hbm.at[pl.ds(base, bpw)], j_idx)

        copies = []
        for c in range(n_chunks):
            sl = pl.ds(c * CHUNK, CHUNK)
            copies.append(pltpu.async_copy(
                uf_hbm.at[u_idx.at[sl]], u_rows.at[sl], sem))
            copies.append(pltpu.async_copy(
                if_hbm.at[i_idx.at[sl]], i_rows.at[sl], sem))
            copies.append(pltpu.async_copy(
                if_hbm.at[j_idx.at[sl]], j_rows.at[sl], sem))
            copies.append(pltpu.async_copy(
                ib_hbm.at[i_idx.at[sl]], bi_v.at[sl], sem))
            copies.append(pltpu.async_copy(
                ib_hbm.at[j_idx.at[sl]], bj_v.at[sl], sem))
        for cp in copies:
            cp.wait()

        lane_iota = lax.iota(jnp.int32, LANES)
        perms = [jnp.bitwise_xor(lane_iota, jnp.full((LANES,), s, jnp.int32))
                 for s in (1, 2, 4, 8)]

        def group_body(g, carry):
            gb = g * LANES
            acc = bi_v[pl.ds(gb, LANES)] - bj_v[pl.ds(gb, LANES)]
            for ee in range(LANES):
                e = gb + ee
                p = None
                for k in range(DIM // LANES):
                    sl = pl.ds(k * LANES, LANES)
                    uv = u_rows[e, sl]
                    dv = i_rows[e, sl] - j_rows[e, sl]
                    t = uv * dv
                    p = t if p is None else p + t
                for perm in perms:
                    p = p + jnp.take(p, perm)
                acc = jnp.where(lane_iota == ee, p + acc, acc)
            out_v[pl.ds(gb, LANES)] = acc
            return carry

        lax.fori_loop(0, bpw // LANES, group_body, 0)

        pltpu.sync_copy(out_v, out_hbm.at[pl.ds(base, bpw)])

    return sc_kernel(u, i, j, user_factors, item_factors,
                     item_biases.reshape(-1))
